# Initial kernel scaffold; baseline (speedup 1.0000x reference)
import jax
import jax.numpy as jnp
from jax.experimental import pallas as pl


def kernel(X, transformation):
    raise NotImplementedError("write your pallas kernel here")


# speedup vs baseline: 2.5822x; 2.5822x over previous
import functools

import jax
import jax.numpy as jnp
from jax import lax
from jax.experimental import pallas as pl
from jax.experimental.pallas import tpu as pltpu
from jax.experimental.pallas import tpu_sc as plsc

H_OUT = 64
W_OUT = 64
DEPTH_N = 65
H_IN = 224
W_IN = 224
C = 16
N_PIX = H_OUT * W_OUT
KPAD = 80
GQ = 72

_XL_OFF = 16
_YL_OFF = 80
_ZL_OFF = 144
_CONSTS_LEN = 224


def _build_sc_call():
    info = plsc.get_sparse_core_info()
    nw = info.num_cores * info.num_subcores
    ppw = N_PIX // nw
    assert ppw * nw == N_PIX
    mesh = plsc.VectorSubcoreMesh(core_axis_name="c", subcore_axis_name="s")

    @functools.partial(
        pl.kernel,
        mesh=mesh,
        compiler_params=pltpu.CompilerParams(
            use_tc_tiling_on_sc=False, needs_layout_passes=False),
        out_type=jax.ShapeDtypeStruct((N_PIX, C), jnp.float32),
        scratch_types=[
            pltpu.VMEM((_CONSTS_LEN,), jnp.float32),
            pltpu.VMEM((2, KPAD), jnp.int32),
            pltpu.VMEM((2, 4, KPAD), jnp.float32),
            pltpu.VMEM((2, GQ, 4 * C), jnp.float32),
            pltpu.VMEM((ppw, C), jnp.float32),
            pltpu.SemaphoreType.DMA,
            pltpu.SemaphoreType.DMA,
        ],
    )
    def sc_fn(quad_hbm, consts_hbm, out_hbm, cv, idx_s, wts_s, rows_s,
              outb_s, sem0, sem1):
        wid = lax.axis_index("s") * info.num_cores + lax.axis_index("c")
        base = wid * ppw
        pltpu.sync_copy(consts_hbm, cv)
        tv = cv[pl.ds(0, 16)]
        t = [tv[q] for q in range(12)]
        sems = (sem0, sem1)
        zeros16i = jnp.zeros((16,), jnp.int32)

        def compute(p, b):
            i = p // W_OUT
            j = p - i * W_OUT
            xlj = plsc.load_gather(cv, [zeros16i + (_XL_OFF + j)])
            yli = plsc.load_gather(cv, [zeros16i + (_YL_OFF + i)])
            sx = t[0] * xlj + t[1] * yli
            sy = t[4] * xlj + t[5] * yli
            sz = t[8] * xlj + t[9] * yli
            for g in range(KPAD // 16):
                zlv = cv[pl.ds(_ZL_OFF + g * 16, 16)]
                x = 0.5 * ((sx + t[2] * zlv + t[3]) + 1.0) * float(W_IN)
                y = 0.5 * ((sy + t[6] * zlv + t[7]) + 1.0) * float(H_IN)
                z = 0.5 * ((sz + t[10] * zlv + t[11]) + 1.0) * float(DEPTH_N)
                x0 = x.astype(jnp.int32)
                y0 = y.astype(jnp.int32)
                z0 = z.astype(jnp.int32)
                x0c = jnp.clip(x0, 0, W_IN - 1)
                x1c = jnp.clip(x0 + 1, 0, W_IN - 1)
                y0c = jnp.clip(y0, 0, H_IN - 1)
                y1c = jnp.clip(y0 + 1, 0, H_IN - 1)
                z0c = jnp.clip(z0, 0, DEPTH_N - 1)
                z1c = jnp.clip(z0 + 1, 0, DEPTH_N - 1)
                x0f = x0c.astype(jnp.float32)
                x1f = x1c.astype(jnp.float32)
                y0f = y0c.astype(jnp.float32)
                y1f = y1c.astype(jnp.float32)
                rw1 = x - x0f
                rw0 = x1f - x
                cw1 = y - y0f
                cw0 = y1f - y
                xcol = x0c == x1c
                ycol = y0c == y1c
                cw0 = jnp.where(xcol, cw0 + cw1, cw0)
                cw1 = jnp.where(xcol, 0.0, cw1)
                rw0 = jnp.where(ycol, rw0 + rw1, rw0)
                rw1 = jnp.where(ycol, 0.0, rw1)
                zw = (jnp.where(z1c == 32, z - z0c.astype(jnp.float32), 0.0)
                      + jnp.where(z0c == 32, z1c.astype(jnp.float32) - z, 0.0))
                a = zw * rw0
                bb = zw * rw1
                sl = pl.ds(g * 16, 16)
                idx_s[b, sl] = y0c * W_IN + x0c
                wts_s[b, 0, sl] = a * cw0
                wts_s[b, 1, sl] = a * cw1
                wts_s[b, 2, sl] = bb * cw0
                wts_s[b, 3, sl] = bb * cw1

        def gather_copy(b):
            return pltpu.make_async_copy(
                quad_hbm.at[idx_s.at[b, pl.ds(0, GQ)]],
                rows_s.at[b], sems[b])

        def accumulate(p, b):
            acc = jnp.zeros((16,), jnp.float32)
            for g in range(KPAD // 16):
                n_l = min(16, DEPTH_N - g * 16)
                sl = pl.ds(g * 16, 16)
                w0v = wts_s[b, 0, sl]
                w1v = wts_s[b, 1, sl]
                w2v = wts_s[b, 2, sl]
                w3v = wts_s[b, 3, sl]
                for l in range(n_l):
                    kk = g * 16 + l
                    r0 = rows_s[b, kk, pl.ds(0, 16)]
                    r1 = rows_s[b, kk, pl.ds(16, 16)]
                    r2 = rows_s[b, kk, pl.ds(32, 16)]
                    r3 = rows_s[b, kk, pl.ds(48, 16)]
                    acc = acc + ((w0v[l] * r0 + w1v[l] * r1)
                                 + (w2v[l] * r2 + w3v[l] * r3))
            outb_s[p - base, :] = acc

        compute(base, 0)
        gather_copy(0).start()

        def body2(it, carry):
            p0 = base + 2 * it
            compute(p0 + 1, 1)
            gather_copy(1).start()
            gather_copy(0).wait()
            accumulate(p0, 0)

            @pl.when(it < ppw // 2 - 1)
            def _():
                compute(p0 + 2, 0)
                gather_copy(0).start()

            gather_copy(1).wait()
            accumulate(p0 + 1, 1)
            return carry

        lax.fori_loop(0, ppw // 2, body2, jnp.int32(0))
        pltpu.sync_copy(outb_s, out_hbm.at[pl.ds(base, ppw)])

    return sc_fn


def kernel(X, transformation):
    B = X.shape[0]
    X0 = X[0]
    xsh = jnp.concatenate([X0[:, 1:, :], X0[:, -1:, :]], axis=1)
    ysh = jnp.concatenate([X0[1:, :, :], X0[-1:, :, :]], axis=0)
    xysh = jnp.concatenate([xsh[1:, :, :], xsh[-1:, :, :]], axis=0)
    quad = jnp.concatenate([X0, xsh, ysh, xysh], axis=-1)
    quad = quad.reshape(H_IN * W_IN, 4 * C)

    xl = jnp.linspace(-1.0, 1.0, W_OUT, dtype=jnp.float32)
    yl = jnp.linspace(-1.0, 1.0, H_OUT, dtype=jnp.float32)
    zl = jnp.linspace(-1.0, 1.0, DEPTH_N, dtype=jnp.float32)
    consts = jnp.concatenate([
        transformation.reshape(12).astype(jnp.float32),
        jnp.zeros((4,), jnp.float32), xl, yl, zl,
        jnp.zeros((_CONSTS_LEN - _ZL_OFF - DEPTH_N,), jnp.float32),
    ])

    out = _build_sc_call()(quad, consts)
    return out.reshape(B, H_OUT, W_OUT, C)


# speedup vs baseline: 11.8506x; 4.5894x over previous
import functools

import jax
import jax.numpy as jnp
from jax import lax
from jax.experimental import pallas as pl
from jax.experimental.pallas import tpu as pltpu
from jax.experimental.pallas import tpu_sc as plsc

H_OUT = 64
W_OUT = 64
DEPTH_N = 65
H_IN = 224
W_IN = 224
C = 16
N_PIX = H_OUT * W_OUT
KPAD = 80

_XL_OFF = 16
_YL_OFF = 80
_CONSTS_LEN = 224


def _build_sc_call():
    info = plsc.get_sparse_core_info()
    nw = info.num_cores * info.num_subcores
    ppw = N_PIX // nw
    assert ppw * nw == N_PIX and ppw % 16 == 0
    mesh = plsc.VectorSubcoreMesh(core_axis_name="c", subcore_axis_name="s")

    @functools.partial(
        pl.kernel,
        mesh=mesh,
        compiler_params=pltpu.CompilerParams(
            use_tc_tiling_on_sc=False, needs_layout_passes=False),
        out_type=jax.ShapeDtypeStruct((N_PIX, C), jnp.float32),
        scratch_types=[
            pltpu.VMEM((_CONSTS_LEN,), jnp.float32),
            pltpu.VMEM((2, KPAD), jnp.int32),
            pltpu.VMEM((2, 4, KPAD), jnp.float32),
            pltpu.VMEM((2, KPAD, 4 * C), jnp.float32),
            pltpu.VMEM((ppw, C), jnp.float32),
            pltpu.SemaphoreType.DMA,
            pltpu.SemaphoreType.DMA,
        ],
    )
    def sc_fn(quad_hbm, consts_hbm, out_hbm, cv, idx_s, wts_s, rows_s,
              outb_s, sem0, sem1):
        wid = lax.axis_index("s") * info.num_cores + lax.axis_index("c")
        pltpu.sync_copy(consts_hbm, cv)
        tv = cv[pl.ds(0, 16)]
        t = [tv[q] for q in range(12)]
        sems = (sem0, sem1)
        lanes = lax.iota(jnp.int32, 16)
        zeros16i = jnp.zeros((16,), jnp.int32)
        zeros16f = jnp.zeros((16,), jnp.float32)

        def zexpr(sz, kv):
            zlv = (kv.astype(jnp.float32) - 32.0) * 0.03125
            return 0.5 * ((sz + t[10] * zlv + t[11]) + 1.0) * float(DEPTH_N)

        def compute_and_start(m, b):
            r = m // 16
            p = (wid + nw * r) * 16 + (m - r * 16)
            i = p // W_OUT
            j = p - i * W_OUT
            xlj = plsc.load_gather(cv, [zeros16i + (_XL_OFF + j)])
            yli = plsc.load_gather(cv, [zeros16i + (_YL_OFF + i)])
            sx = t[0] * xlj + t[1] * yli
            sy = t[4] * xlj + t[5] * yli
            sz = t[8] * xlj + t[9] * yli
            klo = jnp.int32(127)
            khi = jnp.int32(-1)
            for g in range(KPAD // 16):
                kv = g * 16 + lanes
                z = zexpr(sz, kv)
                mband = (z >= 31.0) & (z < 33.0) & (kv < DEPTH_N)
                klo = jnp.minimum(klo, jnp.min(jnp.where(mband, kv, 127)))
                khi = jnp.maximum(khi, jnp.max(jnp.where(mband, kv, -1)))
            c0 = jnp.bitwise_and(klo, -8)
            nc = jnp.where((khi >= klo) & (m < ppw),
                           (khi - c0) // 16 + 1, 0)

            def chunk(ci):
                kv = c0 + ci * 16 + lanes
                zlv = (kv.astype(jnp.float32) - 32.0) * 0.03125
                x = 0.5 * ((sx + t[2] * zlv + t[3]) + 1.0) * float(W_IN)
                y = 0.5 * ((sy + t[6] * zlv + t[7]) + 1.0) * float(H_IN)
                z = 0.5 * ((sz + t[10] * zlv + t[11]) + 1.0) * float(DEPTH_N)
                x0 = x.astype(jnp.int32)
                y0 = y.astype(jnp.int32)
                z0 = z.astype(jnp.int32)
                x0c = jnp.clip(x0, 0, W_IN - 1)
                x1c = jnp.clip(x0 + 1, 0, W_IN - 1)
                y0c = jnp.clip(y0, 0, H_IN - 1)
                y1c = jnp.clip(y0 + 1, 0, H_IN - 1)
                z0c = jnp.clip(z0, 0, DEPTH_N - 1)
                z1c = jnp.clip(z0 + 1, 0, DEPTH_N - 1)
                x0f = x0c.astype(jnp.float32)
                x1f = x1c.astype(jnp.float32)
                y0f = y0c.astype(jnp.float32)
                y1f = y1c.astype(jnp.float32)
                rw1 = x - x0f
                rw0 = x1f - x
                cw1 = y - y0f
                cw0 = y1f - y
                xcol = x0c == x1c
                ycol = y0c == y1c
                cw0 = jnp.where(xcol, cw0 + cw1, cw0)
                cw1 = jnp.where(xcol, 0.0, cw1)
                rw0 = jnp.where(ycol, rw0 + rw1, rw0)
                rw1 = jnp.where(ycol, 0.0, rw1)
                zw = (jnp.where(z1c == 32, z - z0c.astype(jnp.float32), 0.0)
                      + jnp.where(z0c == 32,
                                  z1c.astype(jnp.float32) - z, 0.0))
                zw = jnp.where(kv < DEPTH_N, zw, 0.0)
                a = zw * rw0
                bb = zw * rw1
                sl = pl.ds(ci * 16, 16)
                idx_s[b, sl] = y0c * W_IN + x0c
                wts_s[b, 0, sl] = a * cw0
                wts_s[b, 1, sl] = a * cw1
                wts_s[b, 2, sl] = bb * cw0
                wts_s[b, 3, sl] = bb * cw1

            @pl.when(nc > 0)
            def _():
                chunk(0)
                chunk(1)
                pltpu.async_copy(
                    quad_hbm.at[idx_s.at[b, pl.ds(0, 32)]],
                    rows_s.at[b, pl.ds(0, 32)], sems[b])

            @pl.when(nc > 2)
            def _():
                chunk(2)
                chunk(3)
                chunk(4)
                pltpu.async_copy(
                    quad_hbm.at[idx_s.at[b, pl.ds(32, 48)]],
                    rows_s.at[b, pl.ds(32, 48)], sems[b])

            return nc

        def acc_chunk(b, ci, acc):
            sl = pl.ds(ci * 16, 16)
            w0v = wts_s[b, 0, sl]
            w1v = wts_s[b, 1, sl]
            w2v = wts_s[b, 2, sl]
            w3v = wts_s[b, 3, sl]
            for l in range(16):
                kk = ci * 16 + l
                r0 = rows_s[b, kk, pl.ds(0, 16)]
                r1 = rows_s[b, kk, pl.ds(16, 16)]
                r2 = rows_s[b, kk, pl.ds(32, 16)]
                r3 = rows_s[b, kk, pl.ds(48, 16)]
                acc = acc + ((w0v[l] * r0 + w1v[l] * r1)
                             + (w2v[l] * r2 + w3v[l] * r3))
            return acc

        def drain_acc(m, b, nc):
            outb_s[m, :] = zeros16f

            @pl.when(nc > 0)
            def _():
                pltpu.make_async_copy(
                    quad_hbm.at[idx_s.at[b, pl.ds(0, 32)]],
                    rows_s.at[b, pl.ds(0, 32)], sems[b]).wait()
                acc = acc_chunk(b, 1, acc_chunk(b, 0, zeros16f))
                outb_s[m, :] = acc

            @pl.when(nc > 2)
            def _():
                pltpu.make_async_copy(
                    quad_hbm.at[idx_s.at[b, pl.ds(32, 48)]],
                    rows_s.at[b, pl.ds(32, 48)], sems[b]).wait()
                acc = acc_chunk(b, 4, acc_chunk(b, 3, acc_chunk(b, 2,
                                                                zeros16f)))
                outb_s[m, :] = outb_s[m, :] + acc

        nc0 = compute_and_start(jnp.int32(0), 0)

        def body2(it, nc0):
            nc1 = compute_and_start(2 * it + 1, 1)
            drain_acc(2 * it, 0, nc0)
            nc0n = compute_and_start(2 * it + 2, 0)
            drain_acc(2 * it + 1, 1, nc1)
            return nc0n

        lax.fori_loop(0, ppw // 2, body2, nc0)
        for r in range(ppw // 16):
            pltpu.sync_copy(
                outb_s.at[pl.ds(r * 16, 16)],
                out_hbm.at[pl.ds((wid + nw * r) * 16, 16)])

    return sc_fn


def kernel(X, transformation):
    B = X.shape[0]
    X0 = X[0]
    xsh = jnp.concatenate([X0[:, 1:, :], X0[:, -1:, :]], axis=1)
    ysh = jnp.concatenate([X0[1:, :, :], X0[-1:, :, :]], axis=0)
    xysh = jnp.concatenate([xsh[1:, :, :], xsh[-1:, :, :]], axis=0)
    quad = jnp.concatenate([X0, xsh, ysh, xysh], axis=-1)
    quad = quad.reshape(H_IN * W_IN, 4 * C)

    def bf16r(v):
        i = lax.bitcast_convert_type(v, jnp.int32)
        r = i + jnp.int32(0x7FFF) + jnp.bitwise_and(
            lax.shift_right_logical(i, 16), jnp.int32(1))
        r = jnp.bitwise_and(r, jnp.int32(-65536))
        return lax.bitcast_convert_type(r, jnp.float32)

    xl = bf16r(jnp.linspace(-1.0, 1.0, W_OUT, dtype=jnp.float32))
    yl = bf16r(jnp.linspace(-1.0, 1.0, H_OUT, dtype=jnp.float32))
    consts = jnp.concatenate([
        bf16r(transformation.reshape(12).astype(jnp.float32)),
        jnp.zeros((4,), jnp.float32), xl, yl,
        jnp.zeros((_CONSTS_LEN - _YL_OFF - H_OUT,), jnp.float32),
    ])

    out = _build_sc_call()(quad, consts)
    return out.reshape(B, H_OUT, W_OUT, C)


# speedup vs baseline: 24.6452x; 2.0797x over previous
import functools

import jax
import jax.numpy as jnp
from jax import lax
from jax.experimental import pallas as pl
from jax.experimental.pallas import tpu as pltpu
from jax.experimental.pallas import tpu_sc as plsc

H_OUT = 64
W_OUT = 64
DEPTH_N = 65
H_IN = 224
W_IN = 224
C = 16
N_PIX = H_OUT * W_OUT
KPAD = 80

_XL_OFF = 16
_YL_OFF = 80
_CONSTS_LEN = 224


def _build_sc_call():
    info = plsc.get_sparse_core_info()
    nw = info.num_cores * info.num_subcores
    ppw = N_PIX // nw
    assert ppw * nw == N_PIX and ppw % 16 == 0
    mesh = plsc.VectorSubcoreMesh(core_axis_name="c", subcore_axis_name="s")

    @functools.partial(
        pl.kernel,
        mesh=mesh,
        compiler_params=pltpu.CompilerParams(
            use_tc_tiling_on_sc=False, needs_layout_passes=False),
        out_type=jax.ShapeDtypeStruct((N_PIX, C), jnp.float32),
        scratch_types=[
            pltpu.VMEM((_CONSTS_LEN,), jnp.float32),
            pltpu.VMEM((2, 4 * KPAD), jnp.int32),
            pltpu.VMEM((2, 4, KPAD), jnp.float32),
            pltpu.VMEM((2, 4 * KPAD, C), jnp.float32),
            pltpu.VMEM((ppw, C), jnp.float32),
            pltpu.SemaphoreType.DMA,
            pltpu.SemaphoreType.DMA,
        ],
    )
    def sc_fn(quad_hbm, consts_hbm, out_hbm, cv, idx_s, wts_s, rows_s,
              outb_s, sem0, sem1):
        wid = lax.axis_index("s") * info.num_cores + lax.axis_index("c")
        pltpu.sync_copy(consts_hbm, cv)
        tv = cv[pl.ds(0, 16)]
        t = [tv[q] for q in range(12)]
        sems = (sem0, sem1)
        lanes = lax.iota(jnp.int32, 16)
        zeros16i = jnp.zeros((16,), jnp.int32)
        zeros16f = jnp.zeros((16,), jnp.float32)

        def zexpr(sz, kv):
            zlv = (kv.astype(jnp.float32) - 32.0) * 0.03125
            return 0.5 * ((sz + t[10] * zlv + t[11]) + 1.0) * float(DEPTH_N)

        def compute_and_start(m, b):
            r = m // 16
            p = (wid + nw * r) * 16 + (m - r * 16)
            i = p // W_OUT
            j = p - i * W_OUT
            xlj = plsc.load_gather(cv, [zeros16i + (_XL_OFF + j)])
            yli = plsc.load_gather(cv, [zeros16i + (_YL_OFF + i)])
            sx = t[0] * xlj + t[1] * yli
            sy = t[4] * xlj + t[5] * yli
            sz = t[8] * xlj + t[9] * yli
            klo_v = jnp.full((16,), 127, jnp.int32)
            khi_v = jnp.full((16,), -1, jnp.int32)
            for g in range(KPAD // 16):
                kv = g * 16 + lanes
                z = zexpr(sz, kv)
                mband = (z >= 31.0) & (z < 33.0) & (kv < DEPTH_N)
                klo_v = jnp.minimum(klo_v, jnp.where(mband, kv, 127))
                khi_v = jnp.maximum(khi_v, jnp.where(mband, kv, -1))
            klo = jnp.min(klo_v)
            khi = jnp.max(khi_v)
            c0 = jnp.bitwise_and(klo, -8)
            nc = jnp.where((khi >= klo) & (m < ppw),
                           (khi - c0) // 16 + 1, 0)

            def chunk(ci):
                kv = c0 + ci * 16 + lanes
                zlv = (kv.astype(jnp.float32) - 32.0) * 0.03125
                x = 0.5 * ((sx + t[2] * zlv + t[3]) + 1.0) * float(W_IN)
                y = 0.5 * ((sy + t[6] * zlv + t[7]) + 1.0) * float(H_IN)
                z = 0.5 * ((sz + t[10] * zlv + t[11]) + 1.0) * float(DEPTH_N)
                x0 = x.astype(jnp.int32)
                y0 = y.astype(jnp.int32)
                z0 = z.astype(jnp.int32)
                x0c = jnp.clip(x0, 0, W_IN - 1)
                x1c = jnp.clip(x0 + 1, 0, W_IN - 1)
                y0c = jnp.clip(y0, 0, H_IN - 1)
                y1c = jnp.clip(y0 + 1, 0, H_IN - 1)
                z0c = jnp.clip(z0, 0, DEPTH_N - 1)
                z1c = jnp.clip(z0 + 1, 0, DEPTH_N - 1)
                x0f = x0c.astype(jnp.float32)
                x1f = x1c.astype(jnp.float32)
                y0f = y0c.astype(jnp.float32)
                y1f = y1c.astype(jnp.float32)
                rw1 = x - x0f
                rw0 = x1f - x
                cw1 = y - y0f
                cw0 = y1f - y
                xcol = x0c == x1c
                ycol = y0c == y1c
                cw0 = jnp.where(xcol, cw0 + cw1, cw0)
                cw1 = jnp.where(xcol, 0.0, cw1)
                rw0 = jnp.where(ycol, rw0 + rw1, rw0)
                rw1 = jnp.where(ycol, 0.0, rw1)
                zw = (jnp.where(z1c == 32, z - z0c.astype(jnp.float32), 0.0)
                      + jnp.where(z0c == 32,
                                  z1c.astype(jnp.float32) - z, 0.0))
                zw = jnp.where(kv < DEPTH_N, zw, 0.0)
                a = zw * rw0
                bb = zw * rw1
                x0p = jnp.minimum(x0c + 1, W_IN - 1)
                y0p = jnp.minimum(y0c + 1, H_IN - 1)
                base0 = y0c * W_IN
                base1 = y0p * W_IN
                idx_s[b, pl.ds(ci * 64, 16)] = base0 + x0c
                idx_s[b, pl.ds(ci * 64 + 16, 16)] = base0 + x0p
                idx_s[b, pl.ds(ci * 64 + 32, 16)] = base1 + x0c
                idx_s[b, pl.ds(ci * 64 + 48, 16)] = base1 + x0p
                sl = pl.ds(ci * 16, 16)
                wts_s[b, 0, sl] = a * cw0
                wts_s[b, 1, sl] = a * cw1
                wts_s[b, 2, sl] = bb * cw0
                wts_s[b, 3, sl] = bb * cw1

            @pl.when(nc > 0)
            def _():
                chunk(0)
                pltpu.async_copy(
                    quad_hbm.at[idx_s.at[b, pl.ds(0, 64)]],
                    rows_s.at[b, pl.ds(0, 64)], sems[b])

            @pl.when(nc > 1)
            def _():
                chunk(1)
                pltpu.async_copy(
                    quad_hbm.at[idx_s.at[b, pl.ds(64, 64)]],
                    rows_s.at[b, pl.ds(64, 64)], sems[b])

            @pl.when(nc > 2)
            def _():
                chunk(2)
                chunk(3)
                chunk(4)
                pltpu.async_copy(
                    quad_hbm.at[idx_s.at[b, pl.ds(128, 128)]],
                    rows_s.at[b, pl.ds(128, 128)], sems[b])
                pltpu.async_copy(
                    quad_hbm.at[idx_s.at[b, pl.ds(256, 64)]],
                    rows_s.at[b, pl.ds(256, 64)], sems[b])

            return nc

        def acc_chunk(b, ci, acc):
            sl = pl.ds(ci * 16, 16)
            w0v = wts_s[b, 0, sl]
            w1v = wts_s[b, 1, sl]
            w2v = wts_s[b, 2, sl]
            w3v = wts_s[b, 3, sl]
            for l in range(16):
                r0 = rows_s[b, ci * 64 + l, :]
                r1 = rows_s[b, ci * 64 + 16 + l, :]
                r2 = rows_s[b, ci * 64 + 32 + l, :]
                r3 = rows_s[b, ci * 64 + 48 + l, :]
                acc = acc + ((w0v[l] * r0 + w1v[l] * r1)
                             + (w2v[l] * r2 + w3v[l] * r3))
            return acc

        def drain_acc(m, b, nc):
            outb_s[m, :] = zeros16f

            @pl.when(nc > 0)
            def _():
                pltpu.make_async_copy(
                    quad_hbm.at[idx_s.at[b, pl.ds(0, 64)]],
                    rows_s.at[b, pl.ds(0, 64)], sems[b]).wait()
                outb_s[m, :] = acc_chunk(b, 0, zeros16f)

            @pl.when(nc > 1)
            def _():
                pltpu.make_async_copy(
                    quad_hbm.at[idx_s.at[b, pl.ds(64, 64)]],
                    rows_s.at[b, pl.ds(64, 64)], sems[b]).wait()
                outb_s[m, :] = outb_s[m, :] + acc_chunk(b, 1, zeros16f)

            @pl.when(nc > 2)
            def _():
                pltpu.make_async_copy(
                    quad_hbm.at[idx_s.at[b, pl.ds(128, 128)]],
                    rows_s.at[b, pl.ds(128, 128)], sems[b]).wait()
                pltpu.make_async_copy(
                    quad_hbm.at[idx_s.at[b, pl.ds(256, 64)]],
                    rows_s.at[b, pl.ds(256, 64)], sems[b]).wait()
                acc = acc_chunk(b, 4, acc_chunk(b, 3, acc_chunk(b, 2,
                                                                zeros16f)))
                outb_s[m, :] = outb_s[m, :] + acc

        nc0 = compute_and_start(jnp.int32(0), 0)

        def body2(it, nc0):
            nc1 = compute_and_start(2 * it + 1, 1)
            drain_acc(2 * it, 0, nc0)
            nc0n = compute_and_start(2 * it + 2, 0)
            drain_acc(2 * it + 1, 1, nc1)
            return nc0n

        lax.fori_loop(0, ppw // 2, body2, nc0)
        for r in range(ppw // 16):
            pltpu.sync_copy(
                outb_s.at[pl.ds(r * 16, 16)],
                out_hbm.at[pl.ds((wid + nw * r) * 16, 16)])

    return sc_fn


def kernel(X, transformation):
    B = X.shape[0]
    quad = X.reshape(H_IN * W_IN, C)

    def bf16r(v):
        i = lax.bitcast_convert_type(v, jnp.int32)
        r = i + jnp.int32(0x7FFF) + jnp.bitwise_and(
            lax.shift_right_logical(i, 16), jnp.int32(1))
        r = jnp.bitwise_and(r, jnp.int32(-65536))
        return lax.bitcast_convert_type(r, jnp.float32)

    xl = bf16r(jnp.linspace(-1.0, 1.0, W_OUT, dtype=jnp.float32))
    yl = bf16r(jnp.linspace(-1.0, 1.0, H_OUT, dtype=jnp.float32))
    consts = jnp.concatenate([
        bf16r(transformation.reshape(12).astype(jnp.float32)),
        jnp.zeros((4,), jnp.float32), xl, yl,
        jnp.zeros((_CONSTS_LEN - _YL_OFF - H_OUT,), jnp.float32),
    ])

    out = _build_sc_call()(quad, consts)
    return out.reshape(B, H_OUT, W_OUT, C)
